# Initial kernel scaffold; baseline (speedup 1.0000x reference)
import jax
import jax.numpy as jnp
from jax.experimental import pallas as pl


def kernel(data, table):
    raise NotImplementedError("write your pallas kernel here")


# speedup vs baseline: 1.5476x; 1.5476x over previous
import functools

import jax
import jax.numpy as jnp
from jax import lax
from jax.experimental import pallas as pl
from jax.experimental.pallas import tpu as pltpu
from jax.experimental.pallas import tpu_sc as plsc

EMBED_DIM = 32
BATCH = 16384
FIELDS = 26

NC = 2
NS = 16
NW = NC * NS

B = BATCH * FIELDS
B_PER_W = B // NW
CHUNK = 1024
N_CHUNKS = B_PER_W // CHUNK


def _build():
    mesh = plsc.VectorSubcoreMesh(
        core_axis_name="c", subcore_axis_name="s", num_cores=NC, num_subcores=NS
    )

    @functools.partial(
        pl.kernel,
        mesh=mesh,
        out_type=jax.ShapeDtypeStruct((B, EMBED_DIM), jnp.float32),
        scratch_types=[
            pltpu.VMEM((CHUNK,), jnp.int32),
            pltpu.VMEM((CHUNK, EMBED_DIM), jnp.float32),
            pltpu.SemaphoreType.DMA,
        ],
        compiler_params=pltpu.CompilerParams(use_tc_tiling_on_sc=False),
    )
    def gather_kernel(idx_hbm, table_hbm, out_hbm, idx_v, rows_v, sem):
        wid = lax.axis_index("s") * NC + lax.axis_index("c")
        base = wid * B_PER_W

        def body(c, _):
            off = base + c * CHUNK
            pltpu.sync_copy(idx_hbm.at[pl.ds(off, CHUNK)], idx_v)
            pltpu.async_copy(table_hbm.at[idx_v], rows_v, sem).wait()
            pltpu.sync_copy(rows_v, out_hbm.at[pl.ds(off, CHUNK)])
            return 0

        lax.fori_loop(0, N_CHUNKS, body, 0)

    return gather_kernel


_gather = _build()


@jax.jit
def kernel(data, table):
    flat = data.reshape(-1).astype(jnp.int32)
    out = _gather(flat, table)
    return out.reshape(BATCH, FIELDS, EMBED_DIM)


# speedup vs baseline: 1.5661x; 1.0120x over previous
import functools

import jax
import jax.numpy as jnp
from jax import lax
from jax.experimental import pallas as pl
from jax.experimental.pallas import tpu as pltpu
from jax.experimental.pallas import tpu_sc as plsc

EMBED_DIM = 32
BATCH = 16384
FIELDS = 26

NC = 2
NS = 16
NW = NC * NS

B = BATCH * FIELDS
B_PER_W = B // NW
CHUNK = 1664
N_CHUNKS = B_PER_W // CHUNK


def _build():
    mesh = plsc.VectorSubcoreMesh(
        core_axis_name="c", subcore_axis_name="s", num_cores=NC, num_subcores=NS
    )

    @functools.partial(
        pl.kernel,
        mesh=mesh,
        out_type=jax.ShapeDtypeStruct((B, EMBED_DIM), jnp.float32),
        scratch_types=[
            pltpu.VMEM((CHUNK,), jnp.int32),
            pltpu.VMEM((CHUNK,), jnp.int32),
            pltpu.VMEM((CHUNK, EMBED_DIM), jnp.float32),
            pltpu.VMEM((CHUNK, EMBED_DIM), jnp.float32),
            pltpu.SemaphoreType.DMA,
            pltpu.SemaphoreType.DMA,
            pltpu.SemaphoreType.DMA,
            pltpu.SemaphoreType.DMA,
        ],
        compiler_params=pltpu.CompilerParams(use_tc_tiling_on_sc=False),
    )
    def gather_kernel(idx_hbm, table_hbm, out_hbm, i0, i1, r0, r1, gs0, gs1, os0, os1):
        ibuf, rbuf = [i0, i1], [r0, r1]
        gsem, osem = [gs0, gs1], [os0, os1]
        wid = lax.axis_index("s") * NC + lax.axis_index("c")
        base = wid * B_PER_W

        def off(c):
            return base + c * CHUNK

        pltpu.sync_copy(idx_hbm.at[pl.ds(off(0), CHUNK)], ibuf[0])
        gathers = [pltpu.async_copy(table_hbm.at[ibuf[0]], rbuf[0], gsem[0]), None]
        pltpu.sync_copy(idx_hbm.at[pl.ds(off(1), CHUNK)], ibuf[1])
        outs = [None, None]
        for c in range(N_CHUNKS):
            p, q = c % 2, (c + 1) % 2
            gathers[p].wait()
            if c >= 1:
                outs[q].wait()
            outs[p] = pltpu.async_copy(rbuf[p], out_hbm.at[pl.ds(off(c), CHUNK)], osem[p])
            if c + 1 < N_CHUNKS:
                gathers[q] = pltpu.async_copy(table_hbm.at[ibuf[q]], rbuf[q], gsem[q])
            if c + 2 < N_CHUNKS:
                pltpu.sync_copy(idx_hbm.at[pl.ds(off(c + 2), CHUNK)], ibuf[p])
        outs[(N_CHUNKS - 1) % 2].wait()

    return gather_kernel


_gather = _build()


@jax.jit
def kernel(data, table):
    flat = data.reshape(-1).astype(jnp.int32)
    out = _gather(flat, table)
    return out.reshape(BATCH, FIELDS, EMBED_DIM)
